# Initial kernel scaffold; baseline (speedup 1.0000x reference)
import jax
import jax.numpy as jnp
from jax.experimental import pallas as pl


def kernel(indices, table, W1, b1, W2, b2):
    raise NotImplementedError("write your pallas kernel here")


# speedup vs baseline: 1.1943x; 1.1943x over previous
import functools

import jax
import jax.numpy as jnp
from jax import lax
from jax.experimental import pallas as pl
from jax.experimental.pallas import tpu as pltpu
from jax.experimental.pallas import tpu_sc as plsc

VOCAB = 1000000
EMBED_DIM = 64
AUX_DIM = 128
TARGET_DIM = 64
BATCH = 16384
HIST = 50

TOTAL = BATCH * HIST
IDX_PER_DMA = 128
FIRE = 8
ROWS_PER_GROUP = IDX_PER_DMA * FIRE


def _make_sc_gather():
    info = plsc.get_sparse_core_info()
    nw = info.num_cores * info.num_subcores
    per_w = TOTAL // nw
    dmas_per_w = per_w // IDX_PER_DMA
    groups = dmas_per_w // FIRE
    mesh = plsc.VectorSubcoreMesh(core_axis_name="c", subcore_axis_name="s")

    @functools.partial(
        pl.kernel,
        mesh=mesh,
        out_type=jax.ShapeDtypeStruct((TOTAL, EMBED_DIM), jnp.float32),
        scratch_types=[
            pltpu.VMEM((dmas_per_w, IDX_PER_DMA), jnp.int32),
            pltpu.VMEM((ROWS_PER_GROUP, EMBED_DIM), jnp.float32),
            pltpu.SemaphoreType.DMA,
        ],
        compiler_params=pltpu.CompilerParams(use_tc_tiling_on_sc=False),
    )
    def gather_k(table_hbm, idx_hbm, out_hbm, idx_v, rows_v, sem):
        wid = lax.axis_index("s") * info.num_cores + lax.axis_index("c")
        pltpu.sync_copy(idx_hbm.at[pl.ds(wid * dmas_per_w, dmas_per_w)], idx_v)
        row_base = wid * per_w

        def body(g, carry):
            handles = []
            for b in range(FIRE):
                h = pltpu.async_copy(
                    table_hbm.at[idx_v.at[g * FIRE + b]],
                    rows_v.at[pl.ds(b * IDX_PER_DMA, IDX_PER_DMA)],
                    sem,
                )
                handles.append(h)
            for h in handles:
                h.wait()
            pltpu.sync_copy(
                rows_v,
                out_hbm.at[pl.ds(row_base + g * ROWS_PER_GROUP, ROWS_PER_GROUP)],
            )
            return carry

        lax.fori_loop(0, groups, body, 0)

    return gather_k


_sc_gather = _make_sc_gather()


def _mm_body(x_ref, w1t_ref, b1_ref, w2t_ref, b2_ref, o_ref):
    h = jnp.dot(x_ref[...], w1t_ref[...], preferred_element_type=jnp.float32)
    h = h + b1_ref[...]
    o = jnp.dot(h, w2t_ref[...], preferred_element_type=jnp.float32)
    o_ref[...] = o + b2_ref[...]


def _tc_project(x, w1t, b1, w2t, b2, blk):
    grid = (TOTAL // blk,)
    return pl.pallas_call(
        _mm_body,
        grid=grid,
        in_specs=[
            pl.BlockSpec((blk, EMBED_DIM), lambda i: (i, 0)),
            pl.BlockSpec((EMBED_DIM, AUX_DIM), lambda i: (0, 0)),
            pl.BlockSpec((1, AUX_DIM), lambda i: (0, 0)),
            pl.BlockSpec((AUX_DIM, TARGET_DIM), lambda i: (0, 0)),
            pl.BlockSpec((1, TARGET_DIM), lambda i: (0, 0)),
        ],
        out_specs=pl.BlockSpec((blk, TARGET_DIM), lambda i: (i, 0)),
        out_shape=jax.ShapeDtypeStruct((TOTAL, TARGET_DIM), jnp.float32),
    )(x, w1t, b1, w2t, b2)


def kernel(indices, table, W1, b1, W2, b2):
    idx2d = indices.reshape(-1).astype(jnp.int32).reshape(TOTAL // IDX_PER_DMA, IDX_PER_DMA)
    gathered = _sc_gather(table, idx2d)
    out = _tc_project(
        gathered,
        W1.T,
        b1.reshape(1, AUX_DIM),
        W2.T,
        b2.reshape(1, TARGET_DIM),
        blk=4096,
    )
    return out.reshape(BATCH, HIST, TARGET_DIM)


# speedup vs baseline: 1.6117x; 1.3495x over previous
import functools

import jax
import jax.numpy as jnp
from jax import lax
from jax.experimental import pallas as pl
from jax.experimental.pallas import tpu as pltpu
from jax.experimental.pallas import tpu_sc as plsc

VOCAB = 1000000
EMBED_DIM = 64
AUX_DIM = 128
TARGET_DIM = 64
BATCH = 16384
HIST = 50

TOTAL = BATCH * HIST
IDX_PER_DMA = 128
FIRE = 8
ROWS_PER_GROUP = IDX_PER_DMA * FIRE
NB = 128


def _make_sc_gather():
    info = plsc.get_sparse_core_info()
    nw = info.num_cores * info.num_subcores
    per_w = TOTAL // nw
    dmas_per_w = per_w // IDX_PER_DMA
    groups = dmas_per_w // FIRE
    mesh = plsc.VectorSubcoreMesh(core_axis_name="c", subcore_axis_name="s")

    @functools.partial(
        pl.kernel,
        mesh=mesh,
        out_type=jax.ShapeDtypeStruct((TOTAL, EMBED_DIM), jnp.float32),
        scratch_types=[
            pltpu.VMEM((dmas_per_w, IDX_PER_DMA), jnp.int32),
            pltpu.VMEM((ROWS_PER_GROUP, EMBED_DIM), jnp.float32),
            pltpu.SemaphoreType.DMA,
        ],
        compiler_params=pltpu.CompilerParams(use_tc_tiling_on_sc=False),
    )
    def gather_k(table_hbm, idx_hbm, out_hbm, idx_v, rows_v, sem):
        wid = lax.axis_index("s") * info.num_cores + lax.axis_index("c")
        pltpu.sync_copy(idx_hbm.at[pl.ds(wid * dmas_per_w, dmas_per_w)], idx_v)
        row_base = wid * per_w

        def body(g, carry):
            handles = []
            for b in range(FIRE):
                h = pltpu.async_copy(
                    table_hbm.at[idx_v.at[g * FIRE + b]],
                    rows_v.at[pl.ds(b * IDX_PER_DMA, IDX_PER_DMA)],
                    sem,
                )
                handles.append(h)
            for h in handles:
                h.wait()
            pltpu.sync_copy(
                rows_v,
                out_hbm.at[pl.ds(row_base + g * ROWS_PER_GROUP, ROWS_PER_GROUP)],
            )
            return carry

        lax.fori_loop(0, groups, body, 0)

    return gather_k


_sc_gather = _make_sc_gather()


def _mm_body(x_ref, bd1_ref, bb1_ref, bd2_ref, bb2_ref, o_ref):
    x = x_ref[...]
    h = jnp.dot(x, bd1_ref[...], preferred_element_type=jnp.float32)
    h = h + bb1_ref[...]
    o = jnp.dot(h, bd2_ref[...], preferred_element_type=jnp.float32)
    o_ref[...] = o + bb2_ref[...]


def _tc_project(x128, bd1, bb1, bd2, bb2):
    rows = NB * HIST // 2
    return pl.pallas_call(
        _mm_body,
        grid=(BATCH // NB,),
        in_specs=[
            pl.BlockSpec((rows, 2 * EMBED_DIM), lambda i: (i, 0)),
            pl.BlockSpec((2 * EMBED_DIM, 2 * AUX_DIM), lambda i: (0, 0)),
            pl.BlockSpec((1, 2 * AUX_DIM), lambda i: (0, 0)),
            pl.BlockSpec((2 * AUX_DIM, 2 * TARGET_DIM), lambda i: (0, 0)),
            pl.BlockSpec((1, 2 * TARGET_DIM), lambda i: (0, 0)),
        ],
        out_specs=pl.BlockSpec((rows, 2 * TARGET_DIM), lambda i: (i, 0)),
        out_shape=jax.ShapeDtypeStruct((TOTAL // 2, 2 * TARGET_DIM), jnp.float32),
    )(x128, bd1, bb1, bd2, bb2)


def _block_diag2(w):
    r, c = w.shape
    z = jnp.zeros((r, c), w.dtype)
    return jnp.concatenate(
        [jnp.concatenate([w, z], axis=1), jnp.concatenate([z, w], axis=1)],
        axis=0,
    )


def kernel(indices, table, W1, b1, W2, b2):
    idx2d = indices.reshape(-1).astype(jnp.int32).reshape(TOTAL // IDX_PER_DMA, IDX_PER_DMA)
    gathered = _sc_gather(table, idx2d)
    x128 = gathered.reshape(TOTAL // 2, 2 * EMBED_DIM)
    bd1 = _block_diag2(W1.T)
    bb1 = jnp.tile(b1, 2).reshape(1, 2 * AUX_DIM)
    bd2 = _block_diag2(W2.T)
    bb2 = jnp.tile(b2, 2).reshape(1, 2 * TARGET_DIM)
    out2 = _tc_project(x128, bd1, bb1, bd2, bb2)
    return out2.reshape(BATCH, HIST, TARGET_DIM)
